# baseline (speedup 1.0000x reference)
import jax
import jax.numpy as jnp
import numpy as np
from jax.experimental import pallas as pl
from jax.experimental.pallas import tpu as pltpu

_C = 80
_K = 256
_N = 5000
_NP = 5120
_NCH = _NP // _K
_IMG = 1023.0
_NMS_THR = 0.3
_SCORE_THR = 0.1
_CLIP = 4.135166556742356

_CHN = ("s", "i", "x1", "y1", "x2", "y2")


def _merge_tables():
    alive = list(range(_NCH))
    ai, bi = [], []
    while len(alive) > 1:
        h = len(alive) // 2
        for j in range(h):
            ai.append(alive[j])
            bi.append(alive[j + h])
        alive = alive[:h] + alive[2 * h:]
    return np.array(ai, np.int32), np.array(bi, np.int32)

_MAI, _MBI = _merge_tables()
_NMERGE = len(_MAI)


def _lane_iota(shape):
    return jax.lax.broadcasted_iota(jnp.int32, shape, len(shape) - 1)


def _partner(v, d, lo):
    up = pltpu.roll(v, _K - d, axis=v.ndim - 1)
    dn = pltpu.roll(v, d, axis=v.ndim - 1)
    return jnp.where(lo, up, dn)


def _cmpex(ch, d, desc, lane):
    lo = (lane & d) == 0
    p = {n: _partner(v, d, lo) for n, v in ch.items()}
    a_first = (ch["s"] > p["s"]) | ((ch["s"] == p["s"]) & (ch["i"] < p["i"]))
    keep_winner = desc == lo
    take_self = keep_winner == a_first
    return {n: jnp.where(take_self, ch[n], p[n]) for n in ch}


def _sort256(ch, lane):
    for klog in range(1, 9):
        k = 1 << klog
        desc = (lane & k) == 0
        for dlog in range(klog - 1, -1, -1):
            ch = _cmpex(ch, 1 << dlog, desc, lane)
    return ch


def _xor_flip(ch, lane):
    for dlog in range(8):
        d = 1 << dlog
        lo = (lane & d) == 0
        ch = {n: _partner(v, d, lo) for n, v in ch.items()}
    return ch


def _merge_pair(cha, chbf, lane):
    a_first = (cha["s"] > chbf["s"]) | (
        (cha["s"] == chbf["s"]) & (cha["i"] < chbf["i"])
    )
    ch = {n: jnp.where(a_first, cha[n], chbf[n]) for n in cha}
    true_m = lane >= 0
    for dlog in range(7, -1, -1):
        ch = _cmpex(ch, 1 << dlog, true_m, lane)
    return ch


def _chunk_body(st_ref, dx_ref, dy_ref, dw_ref, dh_ref, r_ref, *out_refs):
    st = st_ref[0]
    m = jnp.max(st, axis=0, keepdims=True)
    e = jnp.exp(st - m)
    p = e / jnp.sum(e, axis=0, keepdims=True)
    fg = p[1:81, :]
    s = jnp.where(fg >= _SCORE_THR, fg, 0.0)

    r = r_ref[0]
    w = r[2:3] - r[0:1]
    h = r[3:4] - r[1:2]
    cx = r[0:1] + 0.5 * w
    cy = r[1:2] + 0.5 * h
    dx = dx_ref[0] * 0.1
    dy = dy_ref[0] * 0.1
    dw = jnp.minimum(dw_ref[0] * 0.2, _CLIP)
    dh = jnp.minimum(dh_ref[0] * 0.2, _CLIP)
    pcx = dx * w + cx
    pcy = dy * h + cy
    pw = jnp.exp(dw) * w
    ph = jnp.exp(dh) * h
    x1 = jnp.clip(pcx - 0.5 * pw, 0.0, _IMG)
    y1 = jnp.clip(pcy - 0.5 * ph, 0.0, _IMG)
    x2 = jnp.clip(pcx + 0.5 * pw, 0.0, _IMG)
    y2 = jnp.clip(pcy + 0.5 * ph, 0.0, _IMG)

    lane = _lane_iota((_C, _K))
    idx = pl.program_id(0) * _K + lane
    ch = {"s": s, "i": idx, "x1": x1, "y1": y1, "x2": x2, "y2": y2}
    ch = _sort256(ch, lane)
    for ref, n in zip(out_refs, _CHN):
        ref[0] = ch[n]


def _phase1(st3, dx3, dy3, dw3, dh3, r3):
    bs = lambda c: pl.BlockSpec((1, c, _K), lambda g: (g, 0, 0))
    out_sd = [
        jax.ShapeDtypeStruct((_NCH, _C, _K), jnp.int32 if n == "i" else jnp.float32)
        for n in _CHN
    ]
    return pl.pallas_call(
        _chunk_body,
        grid=(_NCH,),
        in_specs=[bs(81), bs(_C), bs(_C), bs(_C), bs(_C), bs(4)],
        out_specs=[bs(_C)] * 6,
        out_shape=out_sd,
    )(st3, dx3, dy3, dw3, dh3, r3)


def _tail_body(ai_ref, bi_ref, s_in, i_in, x1_in, y1_in, x2_in, y2_in,
               out_ref, *scratch):
    w_refs = dict(zip(_CHN, scratch[:6]))
    m_ref, keep_ref = scratch[6], scratch[7]
    ins = dict(zip(_CHN, (s_in, i_in, x1_in, y1_in, x2_in, y2_in)))
    for n in _CHN:
        w_refs[n][...] = ins[n][...]

    lane = _lane_iota((_C, _K))

    def mbody(t, carry):
        ai = ai_ref[t]
        bi = bi_ref[t]
        a = {n: w_refs[n][ai] for n in _CHN}
        b = {n: w_refs[n][bi] for n in _CHN}
        bf = _xor_flip(b, lane)
        c = _merge_pair(a, bf, lane)
        for n in _CHN:
            w_refs[n][ai] = c[n]
        return carry

    jax.lax.fori_loop(0, _NMERGE, mbody, 0)

    s = w_refs["s"][0]
    x1 = w_refs["x1"][0]
    y1 = w_refs["y1"][0]
    x2 = w_refs["x2"][0]
    y2 = w_refs["y2"][0]

    area = jnp.maximum(x2 - x1, 0.0) * jnp.maximum(y2 - y1, 0.0)
    blk = 32
    for i0 in range(0, _K, blk):
        sl = slice(i0, i0 + blk)
        ax1 = x1[:, sl][:, :, None]
        ay1 = y1[:, sl][:, :, None]
        ax2 = x2[:, sl][:, :, None]
        ay2 = y2[:, sl][:, :, None]
        aar = area[:, sl][:, :, None]
        xx1 = jnp.maximum(ax1, x1[:, None, :])
        yy1 = jnp.maximum(ay1, y1[:, None, :])
        xx2 = jnp.minimum(ax2, x2[:, None, :])
        yy2 = jnp.minimum(ay2, y2[:, None, :])
        inter = jnp.maximum(xx2 - xx1, 0.0) * jnp.maximum(yy2 - yy1, 0.0)
        iou = inter / (aar + area[:, None, :] - inter + 1e-9)
        lead = jax.lax.broadcasted_iota(jnp.int32, (_C, blk, _K), 1) + i0
        lj = _lane_iota((_C, blk, _K))
        m_ref[:, pl.ds(i0, blk), :] = jnp.where(
            (iou > _NMS_THR) & (lj > lead), 1.0, 0.0)

    keep_ref[...] = jnp.ones((_C, _K), jnp.float32)

    def nbody(i, carry):
        keep = keep_ref[...]
        ki = jnp.sum(jnp.where(lane == i, keep, 0.0), axis=1, keepdims=True)
        row = m_ref[:, pl.ds(i, 1), :].reshape(_C, _K)
        keep_ref[...] = keep * (1.0 - row * ki)
        return carry

    jax.lax.fori_loop(0, _K, nbody, 0)
    s2 = s * keep_ref[...]

    lead2 = jax.lax.broadcasted_iota(jnp.int32, (_C, _K), 0)
    ch = {"s": s2, "i": lead2 * _K + lane,
          "x1": x1, "y1": y1, "x2": x2, "y2": y2}
    ch = _sort256(ch, lane)

    n = _C
    while n > 1:
        h = n // 2
        a = {k: v[:h] for k, v in ch.items()}
        b = {k: v[h: 2 * h] for k, v in ch.items()}
        lane_h = _lane_iota((h, _K))
        bf = _xor_flip(b, lane_h)
        c = _merge_pair(a, bf, lane_h)
        if n > 2 * h:
            ch = {k: jnp.concatenate([c[k], v[2 * h:]], axis=0)
                  for k, v in ch.items()}
        else:
            ch = c
        n = h + (n - 2 * h)

    cls = (ch["i"][0:1] // _K).astype(jnp.float32)
    out_ref[0:1, :] = ch["x1"][0:1]
    out_ref[1:2, :] = ch["y1"][0:1]
    out_ref[2:3, :] = ch["x2"][0:1]
    out_ref[3:4, :] = ch["y2"][0:1]
    out_ref[4:5, :] = ch["s"][0:1]
    out_ref[5:6, :] = cls
    out_ref[6:8, :] = jnp.zeros((2, _K), jnp.float32)


def _tail(chs):
    sd = lambda shape, dt=jnp.float32: pltpu.VMEM(shape, dt)
    return pl.pallas_call(
        _tail_body,
        in_specs=[pl.BlockSpec(memory_space=pltpu.SMEM)] * 2
        + [pl.BlockSpec((_NCH, _C, _K), lambda: (0, 0, 0))] * 6,
        out_specs=pl.BlockSpec((8, _K), lambda: (0, 0)),
        out_shape=jax.ShapeDtypeStruct((8, _K), jnp.float32),
        scratch_shapes=[
            sd((_NCH, _C, _K)), sd((_NCH, _C, _K), jnp.int32),
            sd((_NCH, _C, _K)), sd((_NCH, _C, _K)),
            sd((_NCH, _C, _K)), sd((_NCH, _C, _K)),
            sd((_C, _K, _K)), sd((_C, _K)),
        ],
    )(jnp.asarray(_MAI), jnp.asarray(_MBI), *chs)


@jax.jit
def kernel(batch_rois, bbox_score, bbox_deltas):
    pad = _NP - _N
    sc = bbox_score.reshape(_N, _C + 1)
    sc = jnp.pad(sc, ((0, pad), (0, 0)))
    st3 = jnp.swapaxes(sc.reshape(_NCH, _K, _C + 1), 1, 2)

    d3 = bbox_deltas.reshape(_N, _C + 1, 4)[:, 1:, :]
    d3 = jnp.pad(d3, ((0, pad), (0, 0), (0, 0)))
    d4 = jnp.transpose(d3.reshape(_NCH, _K, _C, 4), (0, 2, 1, 3))
    dx3, dy3, dw3, dh3 = (d4[..., j] for j in range(4))

    r = jnp.pad(batch_rois[0], ((0, pad), (0, 0)))
    r3 = jnp.swapaxes(r.reshape(_NCH, _K, 4), 1, 2)

    chs = _phase1(st3, dx3, dy3, dw3, dh3, r3)
    out = _tail(chs)
    return jnp.transpose(out[0:6, 0:100])[None]


# speedup vs baseline: 1.0359x; 1.0359x over previous
import jax
import jax.numpy as jnp
import numpy as np
from jax.experimental import pallas as pl
from jax.experimental.pallas import tpu as pltpu

_C = 80
_K = 256
_N = 5000
_NP = 5120
_NCH = _NP // _K
_IMG = 1023.0
_NMS_THR = 0.3
_SCORE_THR = 0.1
_CLIP = 4.135166556742356

_CHN = ("s", "i", "x1", "y1", "x2", "y2")


def _merge_tables():
    alive = list(range(_NCH))
    ai, bi = [], []
    while len(alive) > 1:
        h = len(alive) // 2
        for j in range(h):
            ai.append(alive[j])
            bi.append(alive[j + h])
        alive = alive[:h] + alive[2 * h:]
    return np.array(ai, np.int32), np.array(bi, np.int32)

_MAI, _MBI = _merge_tables()
_NMERGE = len(_MAI)


def _lane_iota(shape):
    return jax.lax.broadcasted_iota(jnp.int32, shape, len(shape) - 1)


def _partner(v, d, lo):
    up = pltpu.roll(v, _K - d, axis=v.ndim - 1)
    dn = pltpu.roll(v, d, axis=v.ndim - 1)
    return jnp.where(lo, up, dn)


def _cmpex(ch, d, desc, lane):
    lo = (lane & d) == 0
    p = {n: _partner(v, d, lo) for n, v in ch.items()}
    a_first = (ch["s"] > p["s"]) | ((ch["s"] == p["s"]) & (ch["i"] < p["i"]))
    keep_winner = desc == lo
    take_self = keep_winner == a_first
    return {n: jnp.where(take_self, ch[n], p[n]) for n in ch}


def _sort256(ch, lane):
    for klog in range(1, 9):
        k = 1 << klog
        desc = (lane & k) == 0
        for dlog in range(klog - 1, -1, -1):
            ch = _cmpex(ch, 1 << dlog, desc, lane)
    return ch


def _xor_flip(ch, lane):
    for dlog in range(8):
        d = 1 << dlog
        lo = (lane & d) == 0
        ch = {n: _partner(v, d, lo) for n, v in ch.items()}
    return ch


def _merge_pair(cha, chbf, lane):
    a_first = (cha["s"] > chbf["s"]) | (
        (cha["s"] == chbf["s"]) & (cha["i"] < chbf["i"])
    )
    ch = {n: jnp.where(a_first, cha[n], chbf[n]) for n in cha}
    true_m = lane >= 0
    for dlog in range(7, -1, -1):
        ch = _cmpex(ch, 1 << dlog, true_m, lane)
    return ch


def _chunk_body(st_ref, dx_ref, dy_ref, dw_ref, dh_ref, r_ref, *out_refs):
    st = st_ref[0]
    m = jnp.max(st, axis=0, keepdims=True)
    e = jnp.exp(st - m)
    p = e / jnp.sum(e, axis=0, keepdims=True)
    fg = p[1:81, :]
    s = jnp.where(fg >= _SCORE_THR, fg, 0.0)

    r = r_ref[0]
    w = r[2:3] - r[0:1]
    h = r[3:4] - r[1:2]
    cx = r[0:1] + 0.5 * w
    cy = r[1:2] + 0.5 * h
    dx = dx_ref[0] * 0.1
    dy = dy_ref[0] * 0.1
    dw = jnp.minimum(dw_ref[0] * 0.2, _CLIP)
    dh = jnp.minimum(dh_ref[0] * 0.2, _CLIP)
    pcx = dx * w + cx
    pcy = dy * h + cy
    pw = jnp.exp(dw) * w
    ph = jnp.exp(dh) * h
    x1 = jnp.clip(pcx - 0.5 * pw, 0.0, _IMG)
    y1 = jnp.clip(pcy - 0.5 * ph, 0.0, _IMG)
    x2 = jnp.clip(pcx + 0.5 * pw, 0.0, _IMG)
    y2 = jnp.clip(pcy + 0.5 * ph, 0.0, _IMG)

    lane = _lane_iota((_C, _K))
    idx = pl.program_id(0) * _K + lane
    ch = {"s": s, "i": idx, "x1": x1, "y1": y1, "x2": x2, "y2": y2}
    ch = _sort256(ch, lane)
    for ref, n in zip(out_refs, _CHN):
        ref[0] = ch[n]


def _phase1(st3, dx3, dy3, dw3, dh3, r3):
    bs = lambda c: pl.BlockSpec((1, c, _K), lambda g: (g, 0, 0))
    out_sd = [
        jax.ShapeDtypeStruct((_NCH, _C, _K), jnp.int32 if n == "i" else jnp.float32)
        for n in _CHN
    ]
    return pl.pallas_call(
        _chunk_body,
        grid=(_NCH,),
        in_specs=[bs(81), bs(_C), bs(_C), bs(_C), bs(_C), bs(4)],
        out_specs=[bs(_C)] * 6,
        out_shape=out_sd,
    )(st3, dx3, dy3, dw3, dh3, r3)


def _tail_body(ai_ref, bi_ref, s_in, i_in, x1_in, y1_in, x2_in, y2_in,
               out_ref, *scratch):
    w_refs = dict(zip(_CHN, scratch[:6]))
    m_ref, keep_ref = scratch[6], scratch[7]
    ins = dict(zip(_CHN, (s_in, i_in, x1_in, y1_in, x2_in, y2_in)))
    for n in _CHN:
        w_refs[n][...] = ins[n][...]

    lane = _lane_iota((_C, _K))

    def mbody(t, carry):
        ai = ai_ref[t]
        bi = bi_ref[t]
        a = {n: w_refs[n][ai] for n in _CHN}
        b = {n: w_refs[n][bi] for n in _CHN}
        bf = _xor_flip(b, lane)
        c = _merge_pair(a, bf, lane)
        for n in _CHN:
            w_refs[n][ai] = c[n]
        return carry

    jax.lax.fori_loop(0, _NMERGE, mbody, 0)

    s = w_refs["s"][0]
    x1 = w_refs["x1"][0]
    y1 = w_refs["y1"][0]
    x2 = w_refs["x2"][0]
    y2 = w_refs["y2"][0]

    area = jnp.maximum(x2 - x1, 0.0) * jnp.maximum(y2 - y1, 0.0)
    blk = 32
    for i0 in range(0, _K, blk):
        sl = slice(i0, i0 + blk)
        ax1 = x1[:, sl][:, :, None]
        ay1 = y1[:, sl][:, :, None]
        ax2 = x2[:, sl][:, :, None]
        ay2 = y2[:, sl][:, :, None]
        aar = area[:, sl][:, :, None]
        xx1 = jnp.maximum(ax1, x1[:, None, :])
        yy1 = jnp.maximum(ay1, y1[:, None, :])
        xx2 = jnp.minimum(ax2, x2[:, None, :])
        yy2 = jnp.minimum(ay2, y2[:, None, :])
        inter = jnp.maximum(xx2 - xx1, 0.0) * jnp.maximum(yy2 - yy1, 0.0)
        iou = inter / (aar + area[:, None, :] - inter + 1e-9)
        lead = jax.lax.broadcasted_iota(jnp.int32, (_C, blk, _K), 1) + i0
        lj = _lane_iota((_C, blk, _K))
        m_ref[:, pl.ds(i0, blk), :] = jnp.where(
            (iou > _NMS_THR) & (lj > lead), 1.0, 0.0)

    keep_ref[...] = jnp.ones((_C, _K), jnp.float32)

    npos = jnp.max(jnp.sum(jnp.where(s > 0.0, 1, 0), axis=1))

    def nbody(i, carry):
        keep = keep_ref[...]
        ki = jnp.sum(jnp.where(lane == i, keep, 0.0), axis=1, keepdims=True)
        row = m_ref[:, pl.ds(i, 1), :].reshape(_C, _K)
        keep_ref[...] = keep * (1.0 - row * ki)
        return carry

    jax.lax.fori_loop(0, npos, nbody, 0)
    s2 = s * keep_ref[...]

    lead2 = jax.lax.broadcasted_iota(jnp.int32, (_C, _K), 0)
    ch = {"s": s2, "i": lead2 * _K + lane,
          "x1": x1, "y1": y1, "x2": x2, "y2": y2}
    ch = _sort256(ch, lane)

    n = _C
    while n > 1:
        h = n // 2
        a = {k: v[:h] for k, v in ch.items()}
        b = {k: v[h: 2 * h] for k, v in ch.items()}
        lane_h = _lane_iota((h, _K))
        bf = _xor_flip(b, lane_h)
        c = _merge_pair(a, bf, lane_h)
        if n > 2 * h:
            ch = {k: jnp.concatenate([c[k], v[2 * h:]], axis=0)
                  for k, v in ch.items()}
        else:
            ch = c
        n = h + (n - 2 * h)

    cls = (ch["i"][0:1] // _K).astype(jnp.float32)
    out_ref[0:1, :] = ch["x1"][0:1]
    out_ref[1:2, :] = ch["y1"][0:1]
    out_ref[2:3, :] = ch["x2"][0:1]
    out_ref[3:4, :] = ch["y2"][0:1]
    out_ref[4:5, :] = ch["s"][0:1]
    out_ref[5:6, :] = cls
    out_ref[6:8, :] = jnp.zeros((2, _K), jnp.float32)


def _tail(chs):
    sd = lambda shape, dt=jnp.float32: pltpu.VMEM(shape, dt)
    return pl.pallas_call(
        _tail_body,
        in_specs=[pl.BlockSpec(memory_space=pltpu.SMEM)] * 2
        + [pl.BlockSpec((_NCH, _C, _K), lambda: (0, 0, 0))] * 6,
        out_specs=pl.BlockSpec((8, _K), lambda: (0, 0)),
        out_shape=jax.ShapeDtypeStruct((8, _K), jnp.float32),
        scratch_shapes=[
            sd((_NCH, _C, _K)), sd((_NCH, _C, _K), jnp.int32),
            sd((_NCH, _C, _K)), sd((_NCH, _C, _K)),
            sd((_NCH, _C, _K)), sd((_NCH, _C, _K)),
            sd((_C, _K, _K)), sd((_C, _K)),
        ],
    )(jnp.asarray(_MAI), jnp.asarray(_MBI), *chs)


@jax.jit
def kernel(batch_rois, bbox_score, bbox_deltas):
    pad = _NP - _N
    sc = bbox_score.reshape(_N, _C + 1)
    sc = jnp.pad(sc, ((0, pad), (0, 0)))
    st3 = jnp.swapaxes(sc.reshape(_NCH, _K, _C + 1), 1, 2)

    d3 = bbox_deltas.reshape(_N, _C + 1, 4)[:, 1:, :]
    d3 = jnp.pad(d3, ((0, pad), (0, 0), (0, 0)))
    d4 = jnp.transpose(d3.reshape(_NCH, _K, _C, 4), (0, 2, 1, 3))
    dx3, dy3, dw3, dh3 = (d4[..., j] for j in range(4))

    r = jnp.pad(batch_rois[0], ((0, pad), (0, 0)))
    r3 = jnp.swapaxes(r.reshape(_NCH, _K, 4), 1, 2)

    chs = _phase1(st3, dx3, dy3, dw3, dh3, r3)
    out = _tail(chs)
    return jnp.transpose(out[0:6, 0:100])[None]


# speedup vs baseline: 1.1254x; 1.0864x over previous
import jax
import jax.numpy as jnp
import numpy as np
from jax.experimental import pallas as pl
from jax.experimental.pallas import tpu as pltpu

_C = 80
_K = 256
_N = 5000
_NP = 5120
_NCH = _NP // _K
_IMG = 1023.0
_NMS_THR = 0.3
_SCORE_THR = 0.1
_CLIP = 4.135166556742356

_CHN = ("s", "i", "x1", "y1", "x2", "y2")


def _merge_plan(n):
    alive = list(range(n))
    merges = []
    while len(alive) > 1:
        h = len(alive) // 2
        for j in range(h):
            merges.append([alive[j], alive[j + h]])
        alive = alive[:h] + alive[2 * h:]
    producer = {s: ("init", s) for s in range(n)}
    init_desc = [True] * n
    out_desc = [True] * len(merges)

    def set_dir(src, desc):
        kind, idx = src
        if kind == "init":
            init_desc[idx] = desc
        else:
            out_desc[idx] = desc

    for t, (a, b) in enumerate(merges):
        set_dir(producer[a], True)
        set_dir(producer[b], False)
        producer[a] = ("merge", t)
    return (
        np.array([m[0] for m in merges], np.int32),
        np.array([m[1] for m in merges], np.int32),
        np.array(out_desc, np.int32),
        np.array(init_desc, np.int32),
    )

_MAI, _MBI, _MOD, _MID = _merge_plan(_NCH)
_NMERGE = len(_MAI)
_FAI, _FBI, _FOD, _FID = _merge_plan(_C)


def _prefix_lens(od, n):
    ps, t0 = [], 0
    while n > 1:
        h = n // 2
        seg = od[t0: t0 + h]
        p = int(seg.sum())
        assert (seg == (np.arange(h) < p)).all(), seg
        ps.append(p)
        t0 += h
        n = h + (n - 2 * h)
    return ps

_FP = _prefix_lens(_FOD, _C)
assert (_FID == (np.arange(_C) < int(_FID.sum()))).all()
_FID_P = int(_FID.sum())


def _lane_iota(shape):
    return jax.lax.broadcasted_iota(jnp.int32, shape, len(shape) - 1)


def _partner(v, d, lo):
    up = pltpu.roll(v, _K - d, axis=v.ndim - 1)
    dn = pltpu.roll(v, d, axis=v.ndim - 1)
    return jnp.where(lo, up, dn)


def _cmpex(ch, d, desc, lane):
    lo = (lane & d) == 0
    p = {n: _partner(v, d, lo) for n, v in ch.items()}
    a_first = (ch["s"] > p["s"]) | ((ch["s"] == p["s"]) & (ch["i"] < p["i"]))
    keep_winner = desc == lo
    take_self = keep_winner == a_first
    return {n: jnp.where(take_self, ch[n], p[n]) for n in ch}


def _sort256(ch, lane, wd):
    for klog in range(1, 9):
        k = 1 << klog
        desc = ((lane & k) == 0) == wd
        for dlog in range(klog - 1, -1, -1):
            ch = _cmpex(ch, 1 << dlog, desc, lane)
    return ch


def _merge_pair(cha, chb, lane, wd):
    a_first = (cha["s"] > chb["s"]) | (
        (cha["s"] == chb["s"]) & (cha["i"] < chb["i"])
    )
    ch = {n: jnp.where(a_first, cha[n], chb[n]) for n in cha}
    desc = (lane >= 0) == wd
    for dlog in range(7, -1, -1):
        ch = _cmpex(ch, 1 << dlog, desc, lane)
    return ch


def _chunk_body(mid_ref, st_ref, dx_ref, dy_ref, dw_ref, dh_ref, r_ref, *out_refs):
    st = st_ref[0]
    m = jnp.max(st, axis=0, keepdims=True)
    e = jnp.exp(st - m)
    p = e / jnp.sum(e, axis=0, keepdims=True)
    fg = p[1:81, :]
    s = jnp.where(fg >= _SCORE_THR, fg, 0.0)

    r = r_ref[0]
    w = r[2:3] - r[0:1]
    h = r[3:4] - r[1:2]
    cx = r[0:1] + 0.5 * w
    cy = r[1:2] + 0.5 * h
    dx = dx_ref[0] * 0.1
    dy = dy_ref[0] * 0.1
    dw = jnp.minimum(dw_ref[0] * 0.2, _CLIP)
    dh = jnp.minimum(dh_ref[0] * 0.2, _CLIP)
    pcx = dx * w + cx
    pcy = dy * h + cy
    pw = jnp.exp(dw) * w
    ph = jnp.exp(dh) * h
    x1 = jnp.clip(pcx - 0.5 * pw, 0.0, _IMG)
    y1 = jnp.clip(pcy - 0.5 * ph, 0.0, _IMG)
    x2 = jnp.clip(pcx + 0.5 * pw, 0.0, _IMG)
    y2 = jnp.clip(pcy + 0.5 * ph, 0.0, _IMG)

    lane = _lane_iota((_C, _K))
    idx = pl.program_id(0) * _K + lane
    ch = {"s": s, "i": idx, "x1": x1, "y1": y1, "x2": x2, "y2": y2}
    ch = _sort256(ch, lane, mid_ref[pl.program_id(0)] > 0)
    for ref, n in zip(out_refs, _CHN):
        ref[0] = ch[n]


def _phase1(st3, dx3, dy3, dw3, dh3, r3):
    bs = lambda c: pl.BlockSpec((1, c, _K), lambda g: (g, 0, 0))
    out_sd = [
        jax.ShapeDtypeStruct((_NCH, _C, _K), jnp.int32 if n == "i" else jnp.float32)
        for n in _CHN
    ]
    return pl.pallas_call(
        _chunk_body,
        grid=(_NCH,),
        in_specs=[pl.BlockSpec(memory_space=pltpu.SMEM)]
        + [bs(81), bs(_C), bs(_C), bs(_C), bs(_C), bs(4)],
        out_specs=[bs(_C)] * 6,
        out_shape=out_sd,
    )(jnp.asarray(_MID), st3, dx3, dy3, dw3, dh3, r3)


def _tail_body(ai_ref, bi_ref, od_ref, s_in, i_in, x1_in, y1_in, x2_in, y2_in,
               out_ref, *scratch):
    w_refs = dict(zip(_CHN, scratch[:6]))
    m_ref, keep_ref = scratch[6], scratch[7]
    ins = dict(zip(_CHN, (s_in, i_in, x1_in, y1_in, x2_in, y2_in)))
    for n in _CHN:
        w_refs[n][...] = ins[n][...]

    lane = _lane_iota((_C, _K))

    def mbody(t, carry):
        ai = ai_ref[t]
        bi = bi_ref[t]
        a = {n: w_refs[n][ai] for n in _CHN}
        b = {n: w_refs[n][bi] for n in _CHN}
        c = _merge_pair(a, b, lane, od_ref[t] > 0)
        for n in _CHN:
            w_refs[n][ai] = c[n]
        return carry

    jax.lax.fori_loop(0, _NMERGE, mbody, 0)

    s = w_refs["s"][0]
    x1 = w_refs["x1"][0]
    y1 = w_refs["y1"][0]
    x2 = w_refs["x2"][0]
    y2 = w_refs["y2"][0]

    area = jnp.maximum(x2 - x1, 0.0) * jnp.maximum(y2 - y1, 0.0)
    blk = 32
    for i0 in range(0, _K, blk):
        sl = slice(i0, i0 + blk)
        ax1 = x1[:, sl][:, :, None]
        ay1 = y1[:, sl][:, :, None]
        ax2 = x2[:, sl][:, :, None]
        ay2 = y2[:, sl][:, :, None]
        aar = area[:, sl][:, :, None]
        xx1 = jnp.maximum(ax1, x1[:, None, :])
        yy1 = jnp.maximum(ay1, y1[:, None, :])
        xx2 = jnp.minimum(ax2, x2[:, None, :])
        yy2 = jnp.minimum(ay2, y2[:, None, :])
        inter = jnp.maximum(xx2 - xx1, 0.0) * jnp.maximum(yy2 - yy1, 0.0)
        iou = inter / (aar + area[:, None, :] - inter + 1e-9)
        lead = jax.lax.broadcasted_iota(jnp.int32, (_C, blk, _K), 1) + i0
        lj = _lane_iota((_C, blk, _K))
        m_ref[:, pl.ds(i0, blk), :] = jnp.where(
            (iou > _NMS_THR) & (lj > lead), 1.0, 0.0)

    keep_ref[...] = jnp.ones((_C, _K), jnp.float32)

    npos = jnp.max(jnp.sum(jnp.where(s > 0.0, 1, 0), axis=1))

    def nbody(i, carry):
        keep = keep_ref[...]
        ki = jnp.sum(jnp.where(lane == i, keep, 0.0), axis=1, keepdims=True)
        row = m_ref[:, pl.ds(i, 1), :].reshape(_C, _K)
        keep_ref[...] = keep * (1.0 - row * ki)
        return carry

    jax.lax.fori_loop(0, npos, nbody, 0)
    s2 = s * keep_ref[...]

    lead2 = jax.lax.broadcasted_iota(jnp.int32, (_C, _K), 0)
    ch = {"s": s2, "i": lead2 * _K + lane,
          "x1": x1, "y1": y1, "x2": x2, "y2": y2}
    ch = _sort256(ch, lane, lead2 < _FID_P)

    n = _C
    rnd = 0
    while n > 1:
        h = n // 2
        a = {k: v[:h] for k, v in ch.items()}
        b = {k: v[h: 2 * h] for k, v in ch.items()}
        lane_h = _lane_iota((h, _K))
        wd = jax.lax.broadcasted_iota(jnp.int32, (h, 1), 0) < _FP[rnd]
        c = _merge_pair(a, b, lane_h, wd)
        if n > 2 * h:
            ch = {k: jnp.concatenate([c[k], v[2 * h:]], axis=0)
                  for k, v in ch.items()}
        else:
            ch = c
        rnd += 1
        n = h + (n - 2 * h)

    cls = (ch["i"][0:1] // _K).astype(jnp.float32)
    out_ref[0:1, :] = ch["x1"][0:1]
    out_ref[1:2, :] = ch["y1"][0:1]
    out_ref[2:3, :] = ch["x2"][0:1]
    out_ref[3:4, :] = ch["y2"][0:1]
    out_ref[4:5, :] = ch["s"][0:1]
    out_ref[5:6, :] = cls
    out_ref[6:8, :] = jnp.zeros((2, _K), jnp.float32)


def _tail(chs):
    sd = lambda shape, dt=jnp.float32: pltpu.VMEM(shape, dt)
    return pl.pallas_call(
        _tail_body,
        in_specs=[pl.BlockSpec(memory_space=pltpu.SMEM)] * 3
        + [pl.BlockSpec((_NCH, _C, _K), lambda: (0, 0, 0))] * 6,
        out_specs=pl.BlockSpec((8, _K), lambda: (0, 0)),
        out_shape=jax.ShapeDtypeStruct((8, _K), jnp.float32),
        scratch_shapes=[
            sd((_NCH, _C, _K)), sd((_NCH, _C, _K), jnp.int32),
            sd((_NCH, _C, _K)), sd((_NCH, _C, _K)),
            sd((_NCH, _C, _K)), sd((_NCH, _C, _K)),
            sd((_C, _K, _K)), sd((_C, _K)),
        ],
    )(jnp.asarray(_MAI), jnp.asarray(_MBI), jnp.asarray(_MOD), *chs)


@jax.jit
def kernel(batch_rois, bbox_score, bbox_deltas):
    pad = _NP - _N
    sc = bbox_score.reshape(_N, _C + 1)
    sc = jnp.pad(sc, ((0, pad), (0, 0)))
    st3 = jnp.swapaxes(sc.reshape(_NCH, _K, _C + 1), 1, 2)

    d3 = bbox_deltas.reshape(_N, _C + 1, 4)[:, 1:, :]
    d3 = jnp.pad(d3, ((0, pad), (0, 0), (0, 0)))
    d4 = jnp.transpose(d3.reshape(_NCH, _K, _C, 4), (0, 2, 1, 3))
    dx3, dy3, dw3, dh3 = (d4[..., j] for j in range(4))

    r = jnp.pad(batch_rois[0], ((0, pad), (0, 0)))
    r3 = jnp.swapaxes(r.reshape(_NCH, _K, 4), 1, 2)

    chs = _phase1(st3, dx3, dy3, dw3, dh3, r3)
    out = _tail(chs)
    return jnp.transpose(out[0:6, 0:100])[None]
